# Initial kernel scaffold; baseline (speedup 1.0000x reference)
import jax
import jax.numpy as jnp
from jax.experimental import pallas as pl


def kernel(sentences, sent_lengths, table):
    raise NotImplementedError("write your pallas kernel here")


# speedup vs baseline: 1.1287x; 1.1287x over previous
import functools

import jax
import jax.numpy as jnp
from jax import lax
from jax.experimental import pallas as pl
from jax.experimental.pallas import tpu as pltpu
from jax.experimental.pallas import tpu_sc as plsc

NC = 2
NS = 16
NW = NC * NS
LANES = 16


def _build_emb_kernel(B, L, D, V):
    ROWS = B * L
    RPW = ROWS // NW
    CHUNK = 128
    NCHUNK = RPW // CHUNK
    SENT_PW = B // NW

    mesh = plsc.VectorSubcoreMesh(core_axis_name="c", subcore_axis_name="s")

    @functools.partial(
        pl.kernel,
        out_type=jax.ShapeDtypeStruct((ROWS, D), jnp.float32),
        mesh=mesh,
        compiler_params=pltpu.CompilerParams(
            use_tc_tiling_on_sc=False, needs_layout_passes=False),
        scratch_types=[
            pltpu.VMEM((NCHUNK, CHUNK), jnp.int32),
            pltpu.VMEM((SENT_PW,), jnp.int32),
            pltpu.VMEM((CHUNK,), jnp.float32),
            pltpu.VMEM((CHUNK, D), jnp.float32),
            pltpu.SemaphoreType.DMA,
        ],
    )
    def body(sent_ref, len_ref, table_ref, out_ref, idx_v, lens_v, mask_v,
             rows_v, sem):
        wid = lax.axis_index("s") * NC + lax.axis_index("c")
        row0 = wid * RPW
        pltpu.sync_copy(sent_ref.at[pl.ds(wid * NCHUNK, NCHUNK)], idx_v)
        pltpu.sync_copy(len_ref.at[pl.ds(wid * SENT_PW, SENT_PW)], lens_v)

        shift = 23
        magic = -(-(1 << shift) // L)
        import numpy as _np
        _p = _np.arange(RPW, dtype=_np.int64)
        assert ((_p * magic) >> shift == _p // L).all(), "magic div invalid"

        def chunk_body(j, carry):
            pltpu.async_copy(table_ref.at[idx_v.at[j]], rows_v, sem).wait()
            iota = lax.iota(jnp.int32, LANES)
            for g in range(CHUNK // LANES):
                p = jnp.full((LANES,), j * CHUNK + g * LANES, jnp.int32) + iota
                b_loc = lax.shift_right_logical(
                    p * jnp.int32(magic), jnp.full((LANES,), shift, jnp.int32))
                pos = p - b_loc * jnp.int32(L)
                lenv = plsc.load_gather(lens_v, [b_loc])
                m = (pos < lenv).astype(jnp.float32)
                mask_v[pl.ds(g * LANES, LANES)] = m

            def row_body(r, c2):
                mv = plsc.load_gather(mask_v, [jnp.full((LANES,), r, jnp.int32)])
                for h in range(D // LANES):
                    sl = pl.ds(h * LANES, LANES)
                    rows_v[r, sl] = rows_v[r, sl] * mv
                return c2

            lax.fori_loop(0, CHUNK, row_body, 0)
            pltpu.sync_copy(rows_v, out_ref.at[pl.ds(row0 + j * CHUNK, CHUNK)])
            return carry

        lax.fori_loop(0, NCHUNK, chunk_body, 0)

    return body


def kernel(sentences, sent_lengths, table):
    B, L = sentences.shape
    V, D = table.shape
    sent2d = sentences.reshape(B * L // 128, 128)
    out = _build_emb_kernel(B, L, D, V)(sent2d, sent_lengths, table)
    return out.reshape(B, L, D)


# speedup vs baseline: 1.1766x; 1.0425x over previous
import functools

import numpy as np

import jax
import jax.numpy as jnp
from jax import lax
from jax.experimental import pallas as pl
from jax.experimental.pallas import tpu as pltpu
from jax.experimental.pallas import tpu_sc as plsc

NC = 2
NS = 16
NW = NC * NS
LANES = 16
NBUF = 4


def _build_emb_kernel(B, L, D, V):
    ROWS = B * L
    RPW = ROWS // NW
    CHUNK = 128
    NCHUNK = RPW // CHUNK
    SENT_PW = B // NW

    SHIFT = 23
    MAGIC = -(-(1 << SHIFT) // L)
    _p = np.arange(RPW, dtype=np.int64)
    assert ((_p * MAGIC) >> SHIFT == _p // L).all(), "magic div invalid"

    mesh = plsc.VectorSubcoreMesh(core_axis_name="c", subcore_axis_name="s")

    @functools.partial(
        pl.kernel,
        out_type=jax.ShapeDtypeStruct((ROWS, D), jnp.float32),
        mesh=mesh,
        compiler_params=pltpu.CompilerParams(
            use_tc_tiling_on_sc=False, needs_layout_passes=False),
        scratch_types=[
            pltpu.VMEM((NCHUNK + 2, CHUNK), jnp.int32),
            pltpu.VMEM((SENT_PW,), jnp.int32),
            pltpu.VMEM((CHUNK + LANES,), jnp.float32),
            pltpu.VMEM((NBUF, CHUNK, D), jnp.float32),
        ] + [pltpu.SemaphoreType.DMA] * (2 * NBUF),
    )
    def body(sent_ref, len_ref, table_ref, out_ref, idx_v, lens_v, mask_v,
             rows_v, *sems):
        gsem = sems[:NBUF]
        osem = sems[NBUF:]
        wid = lax.axis_index("s") * NC + lax.axis_index("c")
        row0 = wid * RPW
        pltpu.sync_copy(sent_ref.at[pl.ds(wid * NCHUNK, NCHUNK)],
                        idx_v.at[pl.ds(0, NCHUNK)])
        pltpu.sync_copy(len_ref.at[pl.ds(wid * SENT_PW, SENT_PW)], lens_v)
        zi = jnp.full((LANES,), 0, jnp.int32)
        for k in range(2):
            for g in range(CHUNK // LANES):
                idx_v[NCHUNK + k, pl.ds(g * LANES, LANES)] = zi

        def start_gather(j, b):
            pltpu.async_copy(table_ref.at[idx_v.at[j]], rows_v.at[b],
                             gsem[b])

        def wait_gather(b):
            pltpu.make_async_copy(table_ref.at[idx_v.at[0]], rows_v.at[b],
                                  gsem[b]).wait()

        def start_out(j, b):
            pltpu.async_copy(rows_v.at[b],
                             out_ref.at[pl.ds(row0 + j * CHUNK, CHUNK)],
                             osem[b])

        def wait_out(b):
            pltpu.make_async_copy(rows_v.at[b],
                                  out_ref.at[pl.ds(row0, CHUNK)],
                                  osem[b]).wait()

        def mask_multiply(j, b):
            iota = lax.iota(jnp.int32, LANES)
            for g in range(CHUNK // LANES):
                p = jnp.full((LANES,), j * CHUNK + g * LANES, jnp.int32) + iota
                b_loc = lax.shift_right_logical(
                    p * jnp.int32(MAGIC), jnp.full((LANES,), SHIFT, jnp.int32))
                pos = p - b_loc * jnp.int32(L)
                lenv = plsc.load_gather(lens_v, [b_loc])
                m = (pos < lenv).astype(jnp.float32)
                mask_v[pl.ds(LANES + g * LANES, LANES)] = m
            for r in range(CHUNK):
                mv = plsc.load_gather(mask_v,
                                      [jnp.full((LANES,), LANES + r, jnp.int32)])
                for h in range(D // LANES):
                    sl = pl.ds(h * LANES, LANES)
                    rows_v[b, r, sl] = rows_v[b, r, sl] * mv

        start_gather(0, 0)
        start_gather(1, 1)

        def quad_body(j4, carry):
            for b in range(NBUF):
                j = j4 * NBUF + b
                wait_gather(b)
                mask_multiply(j, b)
                nb = (b + 2) % NBUF
                if b >= 2:
                    wait_out(nb)
                else:
                    @pl.when(j4 > 0)
                    def _():
                        wait_out(nb)

                start_gather(j + 2, nb)
                start_out(j, b)
            return carry

        lax.fori_loop(0, NCHUNK // NBUF, quad_body, 0)
        wait_gather(0)
        wait_gather(1)
        wait_out(2)
        wait_out(3)

    return body


def kernel(sentences, sent_lengths, table):
    B, L = sentences.shape
    V, D = table.shape
    sent2d = sentences.reshape(B * L // 128, 128)
    out = _build_emb_kernel(B, L, D, V)(sent2d, sent_lengths, table)
    return out.reshape(B, L, D)
